# baseline (speedup 1.0000x reference)
import functools

import jax
import jax.numpy as jnp
from jax.experimental import pallas as pl
from jax.experimental.pallas import tpu as pltpu

_BLOCK_ROWS = 512


def _copy_body(in_ref, out_ref):
    out_ref[...] = in_ref[...]


@functools.lru_cache(maxsize=None)
def _make_copy(seq_len: int, hidden: int):
    grid = (seq_len // _BLOCK_ROWS,)
    return pl.pallas_call(
        _copy_body,
        grid=grid,
        in_specs=[pl.BlockSpec((_BLOCK_ROWS, hidden), lambda i: (i, 0))],
        out_specs=pl.BlockSpec((_BLOCK_ROWS, hidden), lambda i: (i, 0)),
        out_shape=jax.ShapeDtypeStruct((seq_len, hidden), jnp.float32),
    )


def kernel(x, emb_table):
    seq_len = x.shape[1]
    hidden = emb_table.shape[1]
    out = _make_copy(seq_len, hidden)(emb_table[:seq_len])
    return out[None]


# speedup vs baseline: 2.0092x; 2.0092x over previous
import functools

import jax
import jax.numpy as jnp
from jax.experimental import pallas as pl
from jax.experimental.pallas import tpu as pltpu

_BLOCK_ROWS = 512


def _copy_body(in_ref, out_ref):
    out_ref[...] = in_ref[...]


@functools.lru_cache(maxsize=None)
def _make_copy(seq_len: int, hidden: int):
    grid = (seq_len // _BLOCK_ROWS,)
    return pl.pallas_call(
        _copy_body,
        grid=grid,
        in_specs=[pl.BlockSpec((_BLOCK_ROWS, hidden), lambda i: (i, 0))],
        out_specs=pl.BlockSpec((_BLOCK_ROWS, hidden), lambda i: (i, 0)),
        out_shape=jax.ShapeDtypeStruct((seq_len, hidden), jnp.float32),
    )


def kernel(x, emb_table):
    seq_len = x.shape[1]
    hidden = emb_table.shape[1]
    out = _make_copy(seq_len, hidden)(emb_table)
    return out[None]
